# baseline (speedup 1.0000x reference)
import functools

import jax
import jax.numpy as jnp
from jax.experimental import pallas as pl

N = 100000
E = 1600000
G = 16
HID = 32

_EDGE_BLK = 12800


def _mlp_blk(u_ref, w_ref, b_ref, o_ref):
    u = jnp.maximum(u_ref[...], 0.0)
    o_ref[...] = jax.lax.dot(u, w_ref[...],
                             preferred_element_type=jnp.float32) + b_ref[...]


def _edge_mlp(u, w, b):
    rows = u.shape[0]
    grid = rows // _EDGE_BLK
    return pl.pallas_call(
        _mlp_blk,
        grid=(grid,),
        in_specs=[
            pl.BlockSpec((_EDGE_BLK, HID), lambda i: (i, 0)),
            pl.BlockSpec((HID, HID), lambda i: (0, 0)),
            pl.BlockSpec((1, HID), lambda i: (0, 0)),
        ],
        out_specs=pl.BlockSpec((_EDGE_BLK, HID), lambda i: (i, 0)),
        out_shape=jax.ShapeDtypeStruct((rows, HID), jnp.float32),
    )(u, w, b.reshape(1, HID))


def kernel(pos, edge_index, batch, W1a, b1a, W1b, b1b, W2a, b2a, W2b, b2b, Wd, bd):
    src = edge_index[0]
    dst = edge_index[1]

    A1 = pos @ (W1a[:3] + W1a[3:6]) + b1a
    B1 = pos @ W1a[3:6]
    u1 = A1[src] - B1[dst]
    m1 = _edge_mlp(u1, W1b, b1b)
    agg1 = jax.ops.segment_max(m1, dst, num_segments=N)
    x = jnp.maximum(jnp.where(jnp.isfinite(agg1), agg1, 0.0), 0.0)

    A2 = x @ W2a[:HID] + pos @ W2a[HID:] + b2a
    B2 = pos @ W2a[HID:]
    u2 = A2[src] - B2[dst]
    m2 = _edge_mlp(u2, W2b, b2b)
    agg2 = jax.ops.segment_max(m2, dst, num_segments=N)
    x2 = jnp.maximum(jnp.where(jnp.isfinite(agg2), agg2, 0.0), 0.0)

    g = jax.ops.segment_max(x2, batch, num_segments=G)
    g = jnp.where(jnp.isfinite(g), g, 0.0)
    logits = g @ Wd + bd
    return logits, jnp.argmax(logits, axis=-1)


# speedup vs baseline: 1.0378x; 1.0378x over previous
import functools

import jax
import jax.numpy as jnp
from jax import lax
from jax.experimental import pallas as pl
from jax.experimental.pallas import tpu as pltpu
from jax.experimental.pallas import tpu_sc as plsc

N = 100000
E = 1600000
G = 16
HID = 32

_EDGE_BLK = 12800

_NC = 2
_NS = 16
_NW = _NC * _NS
_RPW = 3128
_NPAD = _NW * _RPW
_CH = 4000
_GB = 64


def _mlp_blk(u_ref, w_ref, b_ref, o_ref):
    u = jnp.maximum(u_ref[...], 0.0)
    o_ref[...] = jax.lax.dot(u, w_ref[...],
                             preferred_element_type=jnp.float32) + b_ref[...]


def _edge_mlp(u, w, b):
    rows = u.shape[0]
    grid = rows // _EDGE_BLK
    return pl.pallas_call(
        _mlp_blk,
        grid=(grid,),
        in_specs=[
            pl.BlockSpec((_EDGE_BLK, HID), lambda i: (i, 0)),
            pl.BlockSpec((HID, HID), lambda i: (0, 0)),
            pl.BlockSpec((1, HID), lambda i: (0, 0)),
        ],
        out_specs=pl.BlockSpec((_EDGE_BLK, HID), lambda i: (i, 0)),
        out_shape=jax.ShapeDtypeStruct((rows, HID), jnp.float32),
    )(u, w, b.reshape(1, HID))


def _sc_scatter_max_body(m_hbm, dst_hbm, out_hbm,
                         acc, dstc, ids, dstv, rows, sem):
    wid = lax.axis_index("s") * _NC + lax.axis_index("c")
    lo = wid * _RPW
    hi = lo + _RPW
    iota = lax.broadcasted_iota(jnp.int32, (16,), 0)
    zero16 = jnp.zeros((16,), jnp.float32)

    def zacc(i, _):
        acc[i, :] = zero16
        return 0
    lax.fori_loop(0, 2 * _RPW, zacc, 0)

    def iinit(i, _):
        ids[pl.ds(i * 16, 16)] = iota + i * 16
        return 0
    lax.fori_loop(0, (_CH + 16) // 16, iinit, 0)

    def chunk(c, _):
        pltpu.sync_copy(dst_hbm.at[pl.ds(c * _CH, _CH)], dstc)

        def filt(g, off):
            d = dstc[pl.ds(g * 16, 16)]
            msk = (d >= lo) & (d < hi)
            eid = iota + (c * _CH + g * 16)
            cs = plsc.cumsum(msk.astype(jnp.int32))
            pos = off + cs - 1
            plsc.store_scatter(ids, [pos], eid, mask=msk)
            plsc.store_scatter(dstv, [pos], d, mask=msk)
            return off + cs[15]
        n = lax.fori_loop(0, _CH // 16, filt, 0)

        def batch(b, _):
            base = b * _GB
            pltpu.async_copy(m_hbm.at[ids.at[pl.ds(base, _GB)]], rows,
                             sem).wait()
            for g in range(_GB // 16):
                dv = dstv[pl.ds(base + g * 16, 16)]
                for j in range(16):
                    d = dv[j]
                    valid = (base + g * 16 + j < n) & (d >= lo) & (d < hi)

                    @pl.when(valid)
                    def _():
                        r = g * 16 + j
                        a = (d - lo) * 2
                        acc[a, :] = jnp.maximum(acc[a, :],
                                                rows[r, pl.ds(0, 16)])
                        acc[a + 1, :] = jnp.maximum(acc[a + 1, :],
                                                    rows[r, pl.ds(16, 16)])
            return 0
        lax.fori_loop(0, (n + _GB - 1) // _GB, batch, 0)
        return 0

    lax.fori_loop(0, E // _CH, chunk, 0)
    pltpu.sync_copy(acc, out_hbm.at[pl.ds(wid * 2 * _RPW, 2 * _RPW)])


@jax.jit
def _sc_scatter_max(m, dst):
    mesh = plsc.VectorSubcoreMesh(core_axis_name="c", subcore_axis_name="s",
                                  num_cores=_NC, num_subcores=_NS)
    out_ref = jax.new_ref(jnp.zeros((2 * _NPAD, 16), jnp.float32))
    pl.kernel(
        _sc_scatter_max_body,
        mesh=mesh,
        compiler_params=pltpu.CompilerParams(needs_layout_passes=False, use_tc_tiling_on_sc=False),
        scratch_types=[
            pltpu.VMEM((2 * _RPW, 16), jnp.float32),
            pltpu.VMEM((_CH,), jnp.int32),
            pltpu.VMEM((_CH + 16,), jnp.int32),
            pltpu.VMEM((_CH + 16,), jnp.int32),
            pltpu.VMEM((_GB, HID), jnp.float32),
            pltpu.SemaphoreType.DMA,
        ],
    )(m, dst, out_ref)
    return out_ref[...].reshape(_NPAD, HID)[:N]


def kernel(pos, edge_index, batch, W1a, b1a, W1b, b1b, W2a, b2a, W2b, b2b, Wd, bd):
    src = edge_index[0]
    dst = edge_index[1]

    A1 = pos @ (W1a[:3] + W1a[3:6]) + b1a
    B1 = pos @ W1a[3:6]
    u1 = A1[src] - B1[dst]
    m1 = _edge_mlp(u1, W1b, b1b)
    x = _sc_scatter_max(m1, dst)

    A2 = x @ W2a[:HID] + pos @ W2a[HID:] + b2a
    B2 = pos @ W2a[HID:]
    u2 = A2[src] - B2[dst]
    m2 = _edge_mlp(u2, W2b, b2b)
    x2 = _sc_scatter_max(m2, dst)

    g = jax.ops.segment_max(x2, batch, num_segments=G)
    g = jnp.where(jnp.isfinite(g), g, 0.0)
    logits = g @ Wd + bd
    return logits, jnp.argmax(logits, axis=-1)


# speedup vs baseline: 2.3578x; 2.2720x over previous
import functools

import jax
import jax.numpy as jnp
from jax import lax
from jax.experimental import pallas as pl
from jax.experimental.pallas import tpu as pltpu
from jax.experimental.pallas import tpu_sc as plsc

N = 100000
E = 1600000
G = 16
HID = 32
R4 = E // 4

_EDGE_BLK = 3200

_NC = 2
_NS = 16
_NW = _NC * _NS
_RPW = 3128
_NPAD = _NW * _RPW
_CH = 4000
_GB = 64

_EPW = E // _NW
_SB = 1000

_SC_PARAMS = pltpu.CompilerParams(needs_layout_passes=False,
                                  use_tc_tiling_on_sc=False)
_MESH = plsc.VectorSubcoreMesh(core_axis_name="c", subcore_axis_name="s",
                               num_cores=_NC, num_subcores=_NS)


def _mlp_blk(u_ref, w_ref, b_ref, o_ref):
    u = jnp.maximum(u_ref[...], 0.0)
    o_ref[...] = jax.lax.dot(u, w_ref[...],
                             preferred_element_type=jnp.float32) + b_ref[...]


def _edge_mlp4(u4, w, b):
    wbig = jnp.zeros((4 * HID, 4 * HID), jnp.float32)
    for k in range(4):
        wbig = wbig.at[k * HID:(k + 1) * HID, k * HID:(k + 1) * HID].set(w)
    b4 = jnp.tile(b, 4).reshape(1, 4 * HID)
    return pl.pallas_call(
        _mlp_blk,
        grid=(R4 // _EDGE_BLK,),
        in_specs=[
            pl.BlockSpec((_EDGE_BLK, 4 * HID), lambda i: (i, 0)),
            pl.BlockSpec((4 * HID, 4 * HID), lambda i: (0, 0)),
            pl.BlockSpec((1, 4 * HID), lambda i: (0, 0)),
        ],
        out_specs=pl.BlockSpec((_EDGE_BLK, 4 * HID), lambda i: (i, 0)),
        out_shape=jax.ShapeDtypeStruct((R4, 4 * HID), jnp.float32),
    )(u4, wbig, b4)


def _sc_edge_gather_body(a_hbm, b_hbm, src_hbm, dst_hbm, out_hbm,
                         sidx, didx, arows, brows, uv, sema, semb):
    wid = lax.axis_index("s") * _NC + lax.axis_index("c")
    ebase = wid * _EPW

    def chunk(ci, _):
        base = ebase + ci * _SB
        pltpu.sync_copy(src_hbm.at[pl.ds(base, _SB)], sidx)
        pltpu.sync_copy(dst_hbm.at[pl.ds(base, _SB)], didx)
        cpa = pltpu.async_copy(a_hbm.at[sidx], arows, sema)
        cpb = pltpu.async_copy(b_hbm.at[didx], brows, semb)
        cpa.wait()
        cpb.wait()

        def sub(k, _):
            r = k // 2
            c = (k % 2) * 16
            u = arows[r, pl.ds(c, 16)] - brows[r, pl.ds(c, 16)]
            uv[k // 8, pl.ds((k % 8) * 16, 16)] = u
            return 0
        lax.fori_loop(0, _SB * 2, sub, 0)
        pltpu.sync_copy(uv, out_hbm.at[pl.ds(base // 4, _SB // 4)])
        return 0

    lax.fori_loop(0, _EPW // _SB, chunk, 0)


def _sc_edge_gather(a, b, src, dst):
    out_ref = jax.new_ref(jnp.zeros((R4, 4 * HID), jnp.float32))
    pl.kernel(
        _sc_edge_gather_body,
        mesh=_MESH,
        compiler_params=_SC_PARAMS,
        scratch_types=[
            pltpu.VMEM((_SB,), jnp.int32),
            pltpu.VMEM((_SB,), jnp.int32),
            pltpu.VMEM((_SB, HID), jnp.float32),
            pltpu.VMEM((_SB, HID), jnp.float32),
            pltpu.VMEM((_SB // 4, 4 * HID), jnp.float32),
            pltpu.SemaphoreType.DMA,
            pltpu.SemaphoreType.DMA,
        ],
    )(a, b, src, dst, out_ref)
    return out_ref[...]


def _sc_scatter_max_body(m_hbm, dst_hbm, out_hbm,
                         acc, dstc, ids, dstv, rows, sem):
    wid = lax.axis_index("s") * _NC + lax.axis_index("c")
    lo = wid * _RPW
    hi = lo + _RPW
    iota = lax.broadcasted_iota(jnp.int32, (16,), 0)
    zero16 = jnp.zeros((16,), jnp.float32)

    def zacc(i, _):
        acc[i, :] = zero16
        return 0
    lax.fori_loop(0, 2 * _RPW, zacc, 0)

    def iinit(i, _):
        ids[pl.ds(i * 16, 16)] = iota + i * 16
        return 0
    lax.fori_loop(0, (_CH + 16) // 16, iinit, 0)

    def chunk(c, _):
        pltpu.sync_copy(dst_hbm.at[pl.ds(c * _CH, _CH)], dstc)

        def filt(g, off):
            d = dstc[pl.ds(g * 16, 16)]
            msk = (d >= lo) & (d < hi)
            eid = iota + (c * _CH + g * 16)
            cs = plsc.cumsum(msk.astype(jnp.int32))
            pos = off + cs - 1
            plsc.store_scatter(ids, [pos], eid, mask=msk)
            plsc.store_scatter(dstv, [pos], d, mask=msk)
            cnt = plsc.all_reduce_population_count(msk)
            cnt = cnt if cnt.ndim == 0 else cnt[0]
            return off + cnt
        n = lax.fori_loop(0, _CH // 16, filt, 0)

        def batch(b, _):
            base = b * _GB
            pltpu.async_copy(m_hbm.at[ids.at[pl.ds(base, _GB)]], rows,
                             sem).wait()
            for g in range(_GB // 16):
                dv = dstv[pl.ds(base + g * 16, 16)]
                for j in range(16):
                    d = dv[j]
                    valid = (base + g * 16 + j < n) & (d >= lo) & (d < hi)

                    @pl.when(valid)
                    def _():
                        r = g * 16 + j
                        a = (d - lo) * 2
                        acc[a, :] = jnp.maximum(acc[a, :],
                                                rows[r, pl.ds(0, 16)])
                        acc[a + 1, :] = jnp.maximum(acc[a + 1, :],
                                                    rows[r, pl.ds(16, 16)])
            return 0
        lax.fori_loop(0, (n + _GB - 1) // _GB, batch, 0)
        return 0

    lax.fori_loop(0, E // _CH, chunk, 0)
    pltpu.sync_copy(acc, out_hbm.at[pl.ds(wid * 2 * _RPW, 2 * _RPW)])


def _sc_scatter_max(m, dst):
    out_ref = jax.new_ref(jnp.zeros((2 * _NPAD, 16), jnp.float32))
    pl.kernel(
        _sc_scatter_max_body,
        mesh=_MESH,
        compiler_params=_SC_PARAMS,
        scratch_types=[
            pltpu.VMEM((2 * _RPW, 16), jnp.float32),
            pltpu.VMEM((_CH,), jnp.int32),
            pltpu.VMEM((_CH + 16,), jnp.int32),
            pltpu.VMEM((_CH + 16,), jnp.int32),
            pltpu.VMEM((_GB, HID), jnp.float32),
            pltpu.SemaphoreType.DMA,
        ],
    )(m, dst, out_ref)
    return out_ref[...].reshape(_NPAD, HID)[:N]


def _conv_layer(a, b, src, dst, w, bias):
    u4 = _sc_edge_gather(a, b, src, dst)
    m4 = _edge_mlp4(u4, w, bias)
    return _sc_scatter_max(m4.reshape(E, HID), dst)


def kernel(pos, edge_index, batch, W1a, b1a, W1b, b1b, W2a, b2a, W2b, b2b, Wd, bd):
    src = edge_index[0]
    dst = edge_index[1]

    A1 = pos @ (W1a[:3] + W1a[3:6]) + b1a
    B1 = pos @ W1a[3:6]
    x = _conv_layer(A1, B1, src, dst, W1b, b1b)

    A2 = x @ W2a[:HID] + pos @ W2a[HID:] + b2a
    B2 = pos @ W2a[HID:]
    x2 = _conv_layer(A2, B2, src, dst, W2b, b2b)

    g = jax.ops.segment_max(x2, batch, num_segments=G)
    g = jnp.where(jnp.isfinite(g), g, 0.0)
    logits = g @ Wd + bd
    return logits, jnp.argmax(logits, axis=-1)
